# baseline (speedup 1.0000x reference)
import functools

import jax
import jax.numpy as jnp
from jax import lax
from jax.experimental import pallas as pl
from jax.experimental.pallas import tpu as pltpu
from jax.experimental.pallas import tpu_sc as plsc

N_SRC = 10000
N_DST = 2048
E = 320000
D = 128

NC = 2
NS = 16
NW = NC * NS

B_PAD = 10240
GPW = B_PAD // NW
GK = 80

EPW = E // NW
EK = 128
NCH = 79
APW = NCH * EK
E_PAD = NW * APW
TRASH = N_DST
AGR = 2176

HTOT = 12288
SW = HTOT // NS

_mesh = plsc.VectorSubcoreMesh(core_axis_name="c", subcore_axis_name="s")


def _wid():
    return lax.axis_index("s") * NC + lax.axis_index("c")


@functools.partial(
    pl.kernel,
    out_type=[jax.ShapeDtypeStruct((B_PAD, D), jnp.float32),
              jax.ShapeDtypeStruct((HTOT,), jnp.float32),
              jax.ShapeDtypeStruct((HTOT,), jnp.float32)],
    mesh=_mesh,
    scratch_types=[
        pltpu.VMEM((GPW,), jnp.int32),
        [pltpu.VMEM((GK, D), jnp.float32) for _ in range(GPW // GK)],
        [pltpu.SemaphoreType.DMA for _ in range(GPW // GK)],
        pltpu.VMEM((EPW,), jnp.int32),
        pltpu.VMEM((EPW,), jnp.int32),
        pltpu.SemaphoreType.DMA,
        pltpu.SemaphoreType.DMA,
        pltpu.VMEM((HTOT,), jnp.float32),
        pltpu.VMEM((SW,), jnp.float32),
        pltpu.VMEM((SW,), jnp.float32),
        pltpu.VMEM_SHARED((NS * HTOT,), jnp.float32),
    ],
    compiler_params=pltpu.CompilerParams(needs_layout_passes=False),
)
def _gather_hist(x_hbm, nid_hbm, esrc_hbm, edst_hbm,
                 out_hbm, out0_hbm, out1_hbm,
                 gidx_v, rows_bufs, gsems, sidx_v, didx_v, esem0, esem1,
                 hist_v, acc_v, tbuf_v, hist_sh):
    cid = lax.axis_index("c")
    sid = lax.axis_index("s")
    wid = _wid()
    zeros16 = jnp.zeros((16,), jnp.float32)
    ones16 = jnp.ones((16,), jnp.float32)
    nchunk = GPW // GK
    base = wid * GPW

    pltpu.sync_copy(nid_hbm.at[pl.ds(base, GPW)], gidx_v)
    gds = [pltpu.async_copy(x_hbm.at[gidx_v.at[pl.ds(j * GK, GK)]],
                            rows_bufs[j], gsems[j])
           for j in range(nchunk)]
    ed0 = pltpu.async_copy(esrc_hbm.at[pl.ds(wid * EPW, EPW)], sidx_v, esem0)
    ed1 = pltpu.async_copy(edst_hbm.at[pl.ds(wid * EPW, EPW)], didx_v, esem1)

    def zero_body(j, carry):
        hist_v[pl.ds(pl.multiple_of(j * 16, 16), 16)] = zeros16
        return carry

    lax.fori_loop(0, HTOT // 16, zero_body, None)

    for j in range(nchunk):
        gds[j].wait()
        pltpu.sync_copy(rows_bufs[j], out_hbm.at[pl.ds(base + j * GK, GK)])
    ed0.wait()
    ed1.wait()

    def scat_body(i, carry):
        sl = pl.ds(pl.multiple_of(i * 16, 16), 16)
        plsc.addupdate_scatter(hist_v, [sidx_v[sl]], ones16)
        plsc.addupdate_scatter(hist_v, [didx_v[sl] + N_SRC], ones16)
        return carry

    lax.fori_loop(0, EPW // 16, scat_body, None)

    pltpu.sync_copy(hist_v, hist_sh.at[pl.ds(sid * HTOT, HTOT)])
    plsc.subcore_barrier()

    def zacc_body(j, carry):
        acc_v[pl.ds(pl.multiple_of(j * 16, 16), 16)] = zeros16
        return carry

    lax.fori_loop(0, SW // 16, zacc_body, None)

    def red_body(t, carry):
        pltpu.sync_copy(
            hist_sh.at[pl.ds(pl.multiple_of(t * HTOT + sid * SW, 128), SW)],
            tbuf_v)

        def add_body(j, c2):
            sl = pl.ds(pl.multiple_of(j * 16, 16), 16)
            acc_v[sl] = acc_v[sl] + tbuf_v[sl]
            return c2

        lax.fori_loop(0, SW // 16, add_body, None)
        return carry

    lax.fori_loop(0, NS, red_body, None)

    @pl.when(cid == 0)
    def _():
        pltpu.sync_copy(acc_v, out0_hbm.at[pl.ds(sid * SW, SW)])

    @pl.when(cid == 1)
    def _():
        pltpu.sync_copy(acc_v, out1_hbm.at[pl.ds(sid * SW, SW)])


def _matmul_body(x_ref, w_ref, d0_ref, d1_ref, o_ref):
    deg = d0_ref[...] + d1_ref[...]
    f = lax.rsqrt(jnp.maximum(deg, 1.0))
    h = jnp.dot(x_ref[...], w_ref[...], preferred_element_type=jnp.float32,
                precision=lax.Precision.HIGHEST)
    o_ref[...] = h * f


def _matmul(x_g, W, ds0, ds1):
    blk = 512
    grid = B_PAD // blk
    return pl.pallas_call(
        _matmul_body,
        grid=(grid,),
        in_specs=[
            pl.BlockSpec((blk, D), lambda i: (i, 0)),
            pl.BlockSpec((D, D), lambda i: (0, 0)),
            pl.BlockSpec((blk, 1), lambda i: (i, 0)),
            pl.BlockSpec((blk, 1), lambda i: (i, 0)),
        ],
        out_specs=pl.BlockSpec((blk, D), lambda i: (i, 0)),
        out_shape=jax.ShapeDtypeStruct((B_PAD, D), jnp.float32),
    )(x_g, W, ds0, ds1)


@functools.partial(
    pl.kernel,
    out_type=jax.ShapeDtypeStruct((NC, N_DST, D), jnp.float32),
    mesh=_mesh,
    scratch_types=[
        pltpu.VMEM((NCH, EK), jnp.int32),
        pltpu.VMEM((NCH, EK), jnp.int32),
        [pltpu.VMEM((EK, D), jnp.float32) for _ in range(5)],
        [pltpu.SemaphoreType.DMA for _ in range(5)],
        [pltpu.SemaphoreType.DMA for _ in range(5)],
        pltpu.SemaphoreType.DMA,
        pltpu.SemaphoreType.DMA,
        pltpu.VMEM_SHARED((AGR, D), jnp.float32),
    ],
)
def _aggregate(h_hbm, esrc_hbm, edst_hbm, zero_hbm, out_hbm,
               sidx_v, didx_v, rows_bufs, gsems, ssems, isem0, isem1, agg_sh):
    cid = lax.axis_index("c")
    sid = lax.axis_index("s")
    wid = _wid()
    rpt = AGR // NS
    ept = N_DST // NS

    id0 = pltpu.async_copy(esrc_hbm.at[wid], sidx_v, isem0)
    id1 = pltpu.async_copy(edst_hbm.at[wid], didx_v, isem1)
    pltpu.sync_copy(zero_hbm.at[pl.ds(sid * rpt, rpt), :],
                    agg_sh.at[pl.ds(sid * rpt, rpt), :])
    id0.wait()
    id1.wait()
    plsc.subcore_barrier()

    def fire_gather(j, b):
        return pltpu.async_copy(h_hbm.at[sidx_v.at[j]], rows_bufs[b],
                                gsems[b])

    def wait_gather(j, b):
        pltpu.make_async_copy(h_hbm.at[sidx_v.at[j]], rows_bufs[b],
                              gsems[b]).wait()

    def fire_scatter(j, b):
        return pltpu.async_copy(rows_bufs[b], agg_sh.at[didx_v.at[j]],
                                ssems[b], add=True)

    def wait_scatter(j, b):
        pltpu.make_async_copy(rows_bufs[b], agg_sh.at[didx_v.at[j]],
                              ssems[b]).wait()

    for b in range(3):
        fire_gather(b, b)

    NB = 15

    def body(g, carry):
        j0 = 5 * g
        for b in range(2):
            fire_gather(j0 + 3 + b, 3 + b)
        for b in range(3):
            wait_gather(j0 + b, b)
            fire_scatter(j0 + b, b)
        for b in range(3):
            wait_scatter(j0 + b, b)
            fire_gather(j0 + 5 + b, b)
        for b in range(2):
            wait_gather(j0 + 3 + b, 3 + b)
            fire_scatter(j0 + 3 + b, 3 + b)
        for b in range(2):
            wait_scatter(j0 + 3 + b, 3 + b)
        return carry

    lax.fori_loop(0, NB, body, None)
    j0 = 5 * NB
    fire_gather(j0 + 3, 3)
    for b in range(3):
        wait_gather(j0 + b, b)
        fire_scatter(j0 + b, b)
    wait_gather(j0 + 3, 3)
    fire_scatter(j0 + 3, 3)
    for b in range(4):
        wait_scatter(j0 + b, b)

    plsc.subcore_barrier()
    pltpu.sync_copy(agg_sh.at[pl.ds(sid * ept, ept), :],
                    out_hbm.at[cid, pl.ds(sid * ept, ept), :])


def _final_body(p0_ref, p1_ref, d0_ref, d1_ref, b_ref, o_ref):
    g = lax.rsqrt(jnp.maximum(d0_ref[...] + d1_ref[...], 1.0))
    a = (p0_ref[...] + p1_ref[...]) * g + b_ref[...]
    o_ref[...] = jnp.where(a > 0, a, jnp.exp(jnp.minimum(a, 0.0)) - 1.0)


def _finalize(p0, p1, dd0, dd1, b2):
    return pl.pallas_call(
        _final_body,
        grid=(1,),
        in_specs=[
            pl.BlockSpec((N_DST, D), lambda i: (0, 0)),
            pl.BlockSpec((N_DST, D), lambda i: (0, 0)),
            pl.BlockSpec((N_DST, 1), lambda i: (0, 0)),
            pl.BlockSpec((N_DST, 1), lambda i: (0, 0)),
            pl.BlockSpec((1, D), lambda i: (0, 0)),
        ],
        out_specs=pl.BlockSpec((N_DST, D), lambda i: (0, 0)),
        out_shape=jax.ShapeDtypeStruct((N_DST, D), jnp.float32),
    )(p0, p1, dd0, dd1, b2)


def kernel(x, n_id, res_n_id, edge_src, edge_dst, W, b):
    del res_n_id
    nid_pad = jnp.concatenate(
        [n_id, jnp.zeros((B_PAD - N_SRC,), jnp.int32)])
    npad = E_PAD - E
    esrc_r = jnp.concatenate(
        [edge_src, jnp.zeros((npad,), jnp.int32)]).reshape(NW, NCH, EK)
    edst_r = jnp.concatenate(
        [edge_dst, jnp.full((npad,), TRASH, jnp.int32)]).reshape(NW, NCH, EK)

    x_g, hist0, hist1 = _gather_hist(x, nid_pad, edge_src, edge_dst)

    ds0 = hist0[:B_PAD].reshape(B_PAD, 1)
    ds1 = hist1[:B_PAD].reshape(B_PAD, 1)
    h = _matmul(x_g, W, ds0, ds1)

    zeros2d = jnp.zeros((AGR, D), jnp.float32)
    parts = _aggregate(h, esrc_r, edst_r, zeros2d)

    dd0 = hist0[N_SRC:N_SRC + N_DST].reshape(N_DST, 1)
    dd1 = hist1[N_SRC:N_SRC + N_DST].reshape(N_DST, 1)
    return _finalize(parts[0], parts[1], dd0, dd1, b.reshape(1, D))


# speedup vs baseline: 1.0002x; 1.0002x over previous
import functools

import jax
import jax.numpy as jnp
from jax import lax
from jax.experimental import pallas as pl
from jax.experimental.pallas import tpu as pltpu
from jax.experimental.pallas import tpu_sc as plsc

N_SRC = 10000
N_DST = 2048
E = 320000
D = 128

NC = 2
NS = 16
NW = NC * NS

B_PAD = 10240
GPW = B_PAD // NW
GK = 80

EPW = E // NW
EK = 128
NCH = 79
APW = NCH * EK
E_PAD = NW * APW
TRASH = N_DST
AGR = 2176

HTOT = 12288
SW = HTOT // NS

_mesh = plsc.VectorSubcoreMesh(core_axis_name="c", subcore_axis_name="s")


def _wid():
    return lax.axis_index("s") * NC + lax.axis_index("c")


@functools.partial(
    pl.kernel,
    out_type=[jax.ShapeDtypeStruct((B_PAD, D), jnp.float32),
              jax.ShapeDtypeStruct((HTOT,), jnp.float32),
              jax.ShapeDtypeStruct((HTOT,), jnp.float32)],
    mesh=_mesh,
    scratch_types=[
        pltpu.VMEM((GPW,), jnp.int32),
        [pltpu.VMEM((GK, D), jnp.float32) for _ in range(GPW // GK)],
        [pltpu.SemaphoreType.DMA for _ in range(GPW // GK)],
        pltpu.VMEM((EPW,), jnp.int32),
        pltpu.VMEM((EPW,), jnp.int32),
        pltpu.SemaphoreType.DMA,
        pltpu.SemaphoreType.DMA,
        pltpu.VMEM((HTOT,), jnp.float32),
        pltpu.VMEM((SW,), jnp.float32),
        pltpu.VMEM((SW,), jnp.float32),
        pltpu.VMEM_SHARED((NS * HTOT,), jnp.float32),
    ],
    compiler_params=pltpu.CompilerParams(needs_layout_passes=False),
)
def _gather_hist(x_hbm, nid_hbm, esrc_hbm, edst_hbm,
                 out_hbm, out0_hbm, out1_hbm,
                 gidx_v, rows_bufs, gsems, sidx_v, didx_v, esem0, esem1,
                 hist_v, acc_v, tbuf_v, hist_sh):
    cid = lax.axis_index("c")
    sid = lax.axis_index("s")
    wid = _wid()
    zeros16 = jnp.zeros((16,), jnp.float32)
    ones16 = jnp.ones((16,), jnp.float32)
    nchunk = GPW // GK
    base = wid * GPW

    pltpu.sync_copy(nid_hbm.at[pl.ds(base, GPW)], gidx_v)
    gds = [pltpu.async_copy(x_hbm.at[gidx_v.at[pl.ds(j * GK, GK)]],
                            rows_bufs[j], gsems[j])
           for j in range(nchunk)]
    ed0 = pltpu.async_copy(esrc_hbm.at[pl.ds(wid * EPW, EPW)], sidx_v, esem0)
    ed1 = pltpu.async_copy(edst_hbm.at[pl.ds(wid * EPW, EPW)], didx_v, esem1)

    def zero_body(j, carry):
        hist_v[pl.ds(pl.multiple_of(j * 16, 16), 16)] = zeros16
        return carry

    lax.fori_loop(0, HTOT // 16, zero_body, None)

    for j in range(nchunk):
        gds[j].wait()
        pltpu.sync_copy(rows_bufs[j], out_hbm.at[pl.ds(base + j * GK, GK)])
    ed0.wait()
    ed1.wait()

    def scat_body(i, carry):
        sl = pl.ds(pl.multiple_of(i * 16, 16), 16)
        plsc.addupdate_scatter(hist_v, [sidx_v[sl]], ones16)
        plsc.addupdate_scatter(hist_v, [didx_v[sl] + N_SRC], ones16)
        return carry

    lax.fori_loop(0, EPW // 16, scat_body, None)

    pltpu.sync_copy(hist_v, hist_sh.at[pl.ds(sid * HTOT, HTOT)])
    plsc.subcore_barrier()

    def zacc_body(j, carry):
        acc_v[pl.ds(pl.multiple_of(j * 16, 16), 16)] = zeros16
        return carry

    lax.fori_loop(0, SW // 16, zacc_body, None)

    def red_body(t, carry):
        pltpu.sync_copy(
            hist_sh.at[pl.ds(pl.multiple_of(t * HTOT + sid * SW, 128), SW)],
            tbuf_v)

        def add_body(j, c2):
            sl = pl.ds(pl.multiple_of(j * 16, 16), 16)
            acc_v[sl] = acc_v[sl] + tbuf_v[sl]
            return c2

        lax.fori_loop(0, SW // 16, add_body, None)
        return carry

    lax.fori_loop(0, NS, red_body, None)

    @pl.when(cid == 0)
    def _():
        pltpu.sync_copy(acc_v, out0_hbm.at[pl.ds(sid * SW, SW)])

    @pl.when(cid == 1)
    def _():
        pltpu.sync_copy(acc_v, out1_hbm.at[pl.ds(sid * SW, SW)])


def _matmul_body(x_ref, w_ref, d0_ref, d1_ref, o_ref):
    deg = d0_ref[...] + d1_ref[...]
    f = lax.rsqrt(jnp.maximum(deg, 1.0))
    h = jnp.dot(x_ref[...], w_ref[...], preferred_element_type=jnp.float32,
                precision=lax.Precision.HIGHEST)
    o_ref[...] = h * f


def _matmul(x_g, W, ds0, ds1):
    blk = 512
    grid = B_PAD // blk
    return pl.pallas_call(
        _matmul_body,
        grid=(grid,),
        in_specs=[
            pl.BlockSpec((blk, D), lambda i: (i, 0)),
            pl.BlockSpec((D, D), lambda i: (0, 0)),
            pl.BlockSpec((blk, 1), lambda i: (i, 0)),
            pl.BlockSpec((blk, 1), lambda i: (i, 0)),
        ],
        out_specs=pl.BlockSpec((blk, D), lambda i: (i, 0)),
        out_shape=jax.ShapeDtypeStruct((B_PAD, D), jnp.float32),
    )(x_g, W, ds0, ds1)


@functools.partial(
    pl.kernel,
    out_type=jax.ShapeDtypeStruct((NC, N_DST, D), jnp.float32),
    mesh=_mesh,
    scratch_types=[
        pltpu.VMEM((NCH, EK), jnp.int32),
        pltpu.VMEM((NCH, EK), jnp.int32),
        [pltpu.VMEM((EK, D), jnp.float32) for _ in range(5)],
        [pltpu.SemaphoreType.DMA for _ in range(5)],
        [pltpu.SemaphoreType.DMA for _ in range(5)],
        pltpu.SemaphoreType.DMA,
        pltpu.SemaphoreType.DMA,
        pltpu.VMEM_SHARED((AGR, D), jnp.float32),
    ],
)
def _aggregate(h_hbm, esrc_hbm, edst_hbm, zero_hbm, out_hbm,
               sidx_v, didx_v, rows_bufs, gsems, ssems, isem0, isem1, agg_sh):
    cid = lax.axis_index("c")
    sid = lax.axis_index("s")
    wid = _wid()
    rpt = AGR // NS
    ept = N_DST // NS

    id0 = pltpu.async_copy(esrc_hbm.at[wid], sidx_v, isem0)
    id1 = pltpu.async_copy(edst_hbm.at[wid], didx_v, isem1)
    pltpu.sync_copy(zero_hbm.at[pl.ds(sid * rpt, rpt), :],
                    agg_sh.at[pl.ds(sid * rpt, rpt), :])
    id0.wait()
    id1.wait()
    plsc.subcore_barrier()

    def fire_gather(j, b):
        return pltpu.async_copy(h_hbm.at[sidx_v.at[j]], rows_bufs[b],
                                gsems[b])

    def wait_gather(j, b):
        pltpu.make_async_copy(h_hbm.at[sidx_v.at[j]], rows_bufs[b],
                              gsems[b]).wait()

    def fire_scatter(j, b):
        return pltpu.async_copy(rows_bufs[b], agg_sh.at[didx_v.at[j]],
                                ssems[b], add=True)

    def wait_scatter(j, b):
        pltpu.make_async_copy(rows_bufs[b], agg_sh.at[didx_v.at[j]],
                              ssems[b]).wait()

    for b in range(3):
        fire_gather(b, b)

    NB = 15

    def body(g, carry):
        j0 = 5 * g
        for b in range(2):
            fire_gather(j0 + 3 + b, 3 + b)
        for b in range(3):
            wait_gather(j0 + b, b)
            fire_scatter(j0 + b, b)
        for b in range(3):
            wait_scatter(j0 + b, b)
            fire_gather(j0 + 5 + b, b)
        for b in range(2):
            wait_gather(j0 + 3 + b, 3 + b)
            fire_scatter(j0 + 3 + b, 3 + b)
        for b in range(2):
            wait_scatter(j0 + 3 + b, 3 + b)
        return carry

    lax.fori_loop(0, NB, body, None)
    j0 = 5 * NB
    fire_gather(j0 + 3, 3)
    for b in range(3):
        wait_gather(j0 + b, b)
        fire_scatter(j0 + b, b)
    wait_gather(j0 + 3, 3)
    fire_scatter(j0 + 3, 3)
    for b in range(4):
        wait_scatter(j0 + b, b)

    plsc.subcore_barrier()
    pltpu.sync_copy(agg_sh.at[pl.ds(sid * ept, ept), :],
                    out_hbm.at[cid, pl.ds(sid * ept, ept), :])


def _final_body(p0_ref, p1_ref, d0_ref, d1_ref, b_ref, o_ref):
    g = lax.rsqrt(jnp.maximum(d0_ref[...] + d1_ref[...], 1.0))
    a = (p0_ref[...] + p1_ref[...]) * g + b_ref[...]
    o_ref[...] = jnp.where(a > 0, a, jnp.exp(jnp.minimum(a, 0.0)) - 1.0)


def _finalize(p0, p1, dd0, dd1, b2):
    return pl.pallas_call(
        _final_body,
        grid=(1,),
        in_specs=[
            pl.BlockSpec((N_DST, D), lambda i: (0, 0)),
            pl.BlockSpec((N_DST, D), lambda i: (0, 0)),
            pl.BlockSpec((N_DST, 1), lambda i: (0, 0)),
            pl.BlockSpec((N_DST, 1), lambda i: (0, 0)),
            pl.BlockSpec((1, D), lambda i: (0, 0)),
        ],
        out_specs=pl.BlockSpec((N_DST, D), lambda i: (0, 0)),
        out_shape=jax.ShapeDtypeStruct((N_DST, D), jnp.float32),
    )(p0, p1, dd0, dd1, b2)


def kernel(x, n_id, res_n_id, edge_src, edge_dst, W, b):
    del res_n_id
    nid_pad = jnp.concatenate(
        [n_id, jnp.zeros((B_PAD - N_SRC,), jnp.int32)])
    npad = E_PAD - E
    esrc_r = jnp.concatenate(
        [edge_src, jnp.zeros((npad,), jnp.int32)]).reshape(NW, NCH, EK)
    pad_dst = TRASH + (jnp.arange(npad, dtype=jnp.int32) % (AGR - N_DST))
    edst_r = jnp.concatenate(
        [edge_dst, pad_dst]).reshape(NW, NCH, EK)

    x_g, hist0, hist1 = _gather_hist(x, nid_pad, edge_src, edge_dst)

    ds0 = hist0[:B_PAD].reshape(B_PAD, 1)
    ds1 = hist1[:B_PAD].reshape(B_PAD, 1)
    h = _matmul(x_g, W, ds0, ds1)

    zeros2d = jnp.zeros((AGR, D), jnp.float32)
    parts = _aggregate(h, esrc_r, edst_r, zeros2d)

    dd0 = hist0[N_SRC:N_SRC + N_DST].reshape(N_DST, 1)
    dd1 = hist1[N_SRC:N_SRC + N_DST].reshape(N_DST, 1)
    return _finalize(parts[0], parts[1], dd0, dd1, b.reshape(1, D))


# speedup vs baseline: 1.9352x; 1.9349x over previous
import functools

import jax
import jax.numpy as jnp
from jax import lax
from jax.experimental import pallas as pl
from jax.experimental.pallas import tpu as pltpu
from jax.experimental.pallas import tpu_sc as plsc

N_SRC = 10000
N_DST = 2048
E = 320000
D = 128

NC = 2
NS = 16
NW = NC * NS

B_PAD = 10240
GPW = B_PAD // NW
GK = 80

EPW = E // NW
EK = 80
NCH = EPW // EK
AGR = N_DST

HTOT = 12288
SW = HTOT // NS

_mesh = plsc.VectorSubcoreMesh(core_axis_name="c", subcore_axis_name="s")


def _wid():
    return lax.axis_index("s") * NC + lax.axis_index("c")


@functools.partial(
    pl.kernel,
    out_type=[jax.ShapeDtypeStruct((B_PAD, D), jnp.float32),
              jax.ShapeDtypeStruct((HTOT,), jnp.float32),
              jax.ShapeDtypeStruct((HTOT,), jnp.float32)],
    mesh=_mesh,
    scratch_types=[
        pltpu.VMEM((GPW,), jnp.int32),
        [pltpu.VMEM((GK, D), jnp.float32) for _ in range(GPW // GK)],
        [pltpu.SemaphoreType.DMA for _ in range(GPW // GK)],
        pltpu.VMEM((EPW,), jnp.int32),
        pltpu.VMEM((EPW,), jnp.int32),
        pltpu.SemaphoreType.DMA,
        pltpu.SemaphoreType.DMA,
        pltpu.VMEM((HTOT,), jnp.float32),
        pltpu.VMEM((SW,), jnp.float32),
        pltpu.VMEM((SW,), jnp.float32),
        pltpu.VMEM_SHARED((NS * HTOT,), jnp.float32),
    ],
    compiler_params=pltpu.CompilerParams(needs_layout_passes=False),
)
def _gather_hist(x_hbm, nid_hbm, esrc_hbm, edst_hbm,
                 out_hbm, out0_hbm, out1_hbm,
                 gidx_v, rows_bufs, gsems, sidx_v, didx_v, esem0, esem1,
                 hist_v, acc_v, tbuf_v, hist_sh):
    cid = lax.axis_index("c")
    sid = lax.axis_index("s")
    wid = _wid()
    zeros16 = jnp.zeros((16,), jnp.float32)
    ones16 = jnp.ones((16,), jnp.float32)
    nchunk = GPW // GK
    base = wid * GPW

    pltpu.sync_copy(nid_hbm.at[pl.ds(base, GPW)], gidx_v)
    gds = [pltpu.async_copy(x_hbm.at[gidx_v.at[pl.ds(j * GK, GK)]],
                            rows_bufs[j], gsems[j])
           for j in range(nchunk)]
    ed0 = pltpu.async_copy(esrc_hbm.at[pl.ds(wid * EPW, EPW)], sidx_v, esem0)
    ed1 = pltpu.async_copy(edst_hbm.at[pl.ds(wid * EPW, EPW)], didx_v, esem1)

    def zero_body(j, carry):
        hist_v[pl.ds(pl.multiple_of(j * 16, 16), 16)] = zeros16
        return carry

    lax.fori_loop(0, HTOT // 16, zero_body, None)

    for j in range(nchunk):
        gds[j].wait()
        pltpu.sync_copy(rows_bufs[j], out_hbm.at[pl.ds(base + j * GK, GK)])
    ed0.wait()
    ed1.wait()

    def scat_body(i, carry):
        sl = pl.ds(pl.multiple_of(i * 16, 16), 16)
        plsc.addupdate_scatter(hist_v, [sidx_v[sl]], ones16)
        plsc.addupdate_scatter(hist_v, [didx_v[sl] + N_SRC], ones16)
        return carry

    lax.fori_loop(0, EPW // 16, scat_body, None)

    pltpu.sync_copy(hist_v, hist_sh.at[pl.ds(sid * HTOT, HTOT)])
    plsc.subcore_barrier()

    def zacc_body(j, carry):
        acc_v[pl.ds(pl.multiple_of(j * 16, 16), 16)] = zeros16
        return carry

    lax.fori_loop(0, SW // 16, zacc_body, None)

    def red_body(t, carry):
        pltpu.sync_copy(
            hist_sh.at[pl.ds(pl.multiple_of(t * HTOT + sid * SW, 128), SW)],
            tbuf_v)

        def add_body(j, c2):
            sl = pl.ds(pl.multiple_of(j * 16, 16), 16)
            acc_v[sl] = acc_v[sl] + tbuf_v[sl]
            return c2

        lax.fori_loop(0, SW // 16, add_body, None)
        return carry

    lax.fori_loop(0, NS, red_body, None)

    @pl.when(cid == 0)
    def _():
        pltpu.sync_copy(acc_v, out0_hbm.at[pl.ds(sid * SW, SW)])

    @pl.when(cid == 1)
    def _():
        pltpu.sync_copy(acc_v, out1_hbm.at[pl.ds(sid * SW, SW)])


def _matmul_body(x_ref, w_ref, d0_ref, d1_ref, o_ref):
    deg = d0_ref[...] + d1_ref[...]
    f = lax.rsqrt(jnp.maximum(deg, 1.0))
    h = jnp.dot(x_ref[...], w_ref[...], preferred_element_type=jnp.float32,
                precision=lax.Precision.HIGHEST)
    o_ref[...] = h * f


def _matmul(x_g, W, ds0, ds1):
    blk = 512
    grid = B_PAD // blk
    return pl.pallas_call(
        _matmul_body,
        grid=(grid,),
        in_specs=[
            pl.BlockSpec((blk, D), lambda i: (i, 0)),
            pl.BlockSpec((D, D), lambda i: (0, 0)),
            pl.BlockSpec((blk, 1), lambda i: (i, 0)),
            pl.BlockSpec((blk, 1), lambda i: (i, 0)),
        ],
        out_specs=pl.BlockSpec((blk, D), lambda i: (i, 0)),
        out_shape=jax.ShapeDtypeStruct((B_PAD, D), jnp.float32),
    )(x_g, W, ds0, ds1)


@functools.partial(
    pl.kernel,
    out_type=jax.ShapeDtypeStruct((NC, N_DST, D), jnp.float32),
    mesh=_mesh,
    scratch_types=[
        pltpu.VMEM((NCH, EK), jnp.int32),
        pltpu.VMEM((NCH, EK), jnp.int32),
        [pltpu.VMEM((EK, D), jnp.float32) for _ in range(6)],
        [pltpu.SemaphoreType.DMA for _ in range(6)],
        [pltpu.SemaphoreType.DMA for _ in range(6)],
        pltpu.SemaphoreType.DMA,
        pltpu.SemaphoreType.DMA,
        pltpu.VMEM_SHARED((AGR, D), jnp.float32),
    ],
)
def _aggregate(h_hbm, esrc_hbm, edst_hbm, zero_hbm, out_hbm,
               sidx_v, didx_v, rows_bufs, gsems, ssems, isem0, isem1, agg_sh):
    cid = lax.axis_index("c")
    sid = lax.axis_index("s")
    wid = _wid()
    rpt = AGR // NS
    ept = N_DST // NS

    id0 = pltpu.async_copy(esrc_hbm.at[wid], sidx_v, isem0)
    id1 = pltpu.async_copy(edst_hbm.at[wid], didx_v, isem1)
    pltpu.sync_copy(zero_hbm.at[pl.ds(sid * rpt, rpt), :],
                    agg_sh.at[pl.ds(sid * rpt, rpt), :])
    id0.wait()
    id1.wait()
    plsc.subcore_barrier()

    def fire_gather(j, b):
        return pltpu.async_copy(h_hbm.at[sidx_v.at[j]], rows_bufs[b],
                                gsems[b])

    def wait_gather(j, b):
        pltpu.make_async_copy(h_hbm.at[sidx_v.at[j]], rows_bufs[b],
                              gsems[b]).wait()

    def fire_scatter(j, b):
        return pltpu.async_copy(rows_bufs[b], agg_sh.at[didx_v.at[j]],
                                ssems[b], add=True)

    def wait_scatter(j, b):
        pltpu.make_async_copy(rows_bufs[b], agg_sh.at[didx_v.at[j]],
                              ssems[b]).wait()

    for b in range(3):
        fire_gather(b, b)

    NB = 20

    def body(g, carry):
        j0 = 6 * g
        for b in range(3):
            fire_gather(j0 + 3 + b, 3 + b)
        for b in range(3):
            wait_gather(j0 + b, b)
            fire_scatter(j0 + b, b)
        for b in range(3):
            wait_scatter(j0 + b, b)
            fire_gather(j0 + 6 + b, b)
        for b in range(3):
            wait_gather(j0 + 3 + b, 3 + b)
            fire_scatter(j0 + 3 + b, 3 + b)
        for b in range(3):
            wait_scatter(j0 + 3 + b, 3 + b)
        return carry

    lax.fori_loop(0, NB, body, None)
    j0 = 6 * NB
    for b in range(3):
        wait_gather(j0 + b, b)
        fire_scatter(j0 + b, b)
    fire_gather(j0 + 3, 3)
    fire_gather(j0 + 4, 4)
    for b in range(2):
        wait_gather(j0 + 3 + b, 3 + b)
        fire_scatter(j0 + 3 + b, 3 + b)
    for b in range(5):
        wait_scatter(j0 + b, b)

    plsc.subcore_barrier()
    pltpu.sync_copy(agg_sh.at[pl.ds(sid * ept, ept), :],
                    out_hbm.at[cid, pl.ds(sid * ept, ept), :])


def _final_body(p0_ref, p1_ref, d0_ref, d1_ref, b_ref, o_ref):
    g = lax.rsqrt(jnp.maximum(d0_ref[...] + d1_ref[...], 1.0))
    a = (p0_ref[...] + p1_ref[...]) * g + b_ref[...]
    o_ref[...] = jnp.where(a > 0, a, jnp.exp(jnp.minimum(a, 0.0)) - 1.0)


def _finalize(p0, p1, dd0, dd1, b2):
    return pl.pallas_call(
        _final_body,
        grid=(1,),
        in_specs=[
            pl.BlockSpec((N_DST, D), lambda i: (0, 0)),
            pl.BlockSpec((N_DST, D), lambda i: (0, 0)),
            pl.BlockSpec((N_DST, 1), lambda i: (0, 0)),
            pl.BlockSpec((N_DST, 1), lambda i: (0, 0)),
            pl.BlockSpec((1, D), lambda i: (0, 0)),
        ],
        out_specs=pl.BlockSpec((N_DST, D), lambda i: (0, 0)),
        out_shape=jax.ShapeDtypeStruct((N_DST, D), jnp.float32),
    )(p0, p1, dd0, dd1, b2)


def kernel(x, n_id, res_n_id, edge_src, edge_dst, W, b):
    del res_n_id
    nid_pad = jnp.concatenate(
        [n_id, jnp.zeros((B_PAD - N_SRC,), jnp.int32)])
    esrc_r = edge_src.reshape(NW, NCH, EK)
    edst_r = edge_dst.reshape(NW, NCH, EK)

    x_g, hist0, hist1 = _gather_hist(x, nid_pad, edge_src, edge_dst)

    ds0 = hist0[:B_PAD].reshape(B_PAD, 1)
    ds1 = hist1[:B_PAD].reshape(B_PAD, 1)
    h = _matmul(x_g, W, ds0, ds1)

    zeros2d = jnp.zeros((AGR, D), jnp.float32)
    parts = _aggregate(h, esrc_r, edst_r, zeros2d)

    dd0 = hist0[N_SRC:N_SRC + N_DST].reshape(N_DST, 1)
    dd1 = hist1[N_SRC:N_SRC + N_DST].reshape(N_DST, 1)
    return _finalize(parts[0], parts[1], dd0, dd1, b.reshape(1, D))


# speedup vs baseline: 1.9743x; 1.0202x over previous
import functools

import jax
import jax.numpy as jnp
from jax import lax
from jax.experimental import pallas as pl
from jax.experimental.pallas import tpu as pltpu
from jax.experimental.pallas import tpu_sc as plsc

N_SRC = 10000
N_DST = 2048
E = 320000
D = 128

NC = 2
NS = 16
NW = NC * NS

B_PAD = 10240
GPW = B_PAD // NW
GK = 80

EPW = E // NW
EK = 80
NCH = EPW // EK
AGR = N_DST

HTOT = 12288
SW = HTOT // NS

_mesh = plsc.VectorSubcoreMesh(core_axis_name="c", subcore_axis_name="s")


def _wid():
    return lax.axis_index("s") * NC + lax.axis_index("c")


@functools.partial(
    pl.kernel,
    out_type=[jax.ShapeDtypeStruct((B_PAD, D), jnp.float32),
              jax.ShapeDtypeStruct((HTOT,), jnp.float32),
              jax.ShapeDtypeStruct((HTOT,), jnp.float32)],
    mesh=_mesh,
    scratch_types=[
        pltpu.VMEM((GPW,), jnp.int32),
        [pltpu.VMEM((GK, D), jnp.float32) for _ in range(GPW // GK)],
        [pltpu.SemaphoreType.DMA for _ in range(GPW // GK)],
        pltpu.VMEM((EPW,), jnp.int32),
        pltpu.VMEM((EPW,), jnp.int32),
        pltpu.SemaphoreType.DMA,
        pltpu.SemaphoreType.DMA,
        pltpu.VMEM((HTOT,), jnp.float32),
        pltpu.VMEM((SW,), jnp.float32),
        pltpu.VMEM((NS, SW), jnp.float32),
        pltpu.VMEM_SHARED((NS, HTOT), jnp.float32),
    ],
    compiler_params=pltpu.CompilerParams(needs_layout_passes=False),
)
def _gather_hist(x_hbm, nid_hbm, esrc_hbm, edst_hbm,
                 out_hbm, out0_hbm, out1_hbm,
                 gidx_v, rows_bufs, gsems, sidx_v, didx_v, esem0, esem1,
                 hist_v, acc_v, tbuf_v, hist_sh):
    cid = lax.axis_index("c")
    sid = lax.axis_index("s")
    wid = _wid()
    zeros16 = jnp.zeros((16,), jnp.float32)
    ones16 = jnp.ones((16,), jnp.float32)
    nchunk = GPW // GK
    base = wid * GPW

    pltpu.sync_copy(nid_hbm.at[pl.ds(base, GPW)], gidx_v)
    gds = [pltpu.async_copy(x_hbm.at[gidx_v.at[pl.ds(j * GK, GK)]],
                            rows_bufs[j], gsems[j])
           for j in range(nchunk)]
    ed0 = pltpu.async_copy(esrc_hbm.at[pl.ds(wid * EPW, EPW)], sidx_v, esem0)
    ed1 = pltpu.async_copy(edst_hbm.at[pl.ds(wid * EPW, EPW)], didx_v, esem1)

    def zero_body(j, carry):
        hist_v[pl.ds(pl.multiple_of(j * 16, 16), 16)] = zeros16
        return carry

    lax.fori_loop(0, HTOT // 16, zero_body, None)

    for j in range(nchunk):
        gds[j].wait()
        pltpu.sync_copy(rows_bufs[j], out_hbm.at[pl.ds(base + j * GK, GK)])
    ed0.wait()
    ed1.wait()

    def scat_body(i, carry):
        sl = pl.ds(pl.multiple_of(i * 16, 16), 16)
        plsc.addupdate_scatter(hist_v, [sidx_v[sl]], ones16)
        plsc.addupdate_scatter(hist_v, [didx_v[sl] + N_SRC], ones16)
        return carry

    lax.fori_loop(0, EPW // 16, scat_body, None)

    pltpu.sync_copy(hist_v, hist_sh.at[sid])
    plsc.subcore_barrier()
    pltpu.sync_copy(hist_sh.at[:, pl.ds(sid * SW, SW)], tbuf_v)

    def red_body(j, carry):
        sl = pl.ds(pl.multiple_of(j * 16, 16), 16)
        a = tbuf_v[0, sl]
        for t in range(1, NS):
            a = a + tbuf_v[t, sl]
        acc_v[sl] = a
        return carry

    lax.fori_loop(0, SW // 16, red_body, None)

    @pl.when(cid == 0)
    def _():
        pltpu.sync_copy(acc_v, out0_hbm.at[pl.ds(sid * SW, SW)])

    @pl.when(cid == 1)
    def _():
        pltpu.sync_copy(acc_v, out1_hbm.at[pl.ds(sid * SW, SW)])


def _matmul_body(x_ref, w_ref, d0_ref, d1_ref, o_ref):
    deg = d0_ref[...] + d1_ref[...]
    f = lax.rsqrt(jnp.maximum(deg, 1.0))
    h = jnp.dot(x_ref[...], w_ref[...], preferred_element_type=jnp.float32,
                precision=lax.Precision.HIGHEST)
    o_ref[...] = h * f


def _matmul(x_g, W, ds0, ds1):
    blk = 512
    grid = B_PAD // blk
    return pl.pallas_call(
        _matmul_body,
        grid=(grid,),
        in_specs=[
            pl.BlockSpec((blk, D), lambda i: (i, 0)),
            pl.BlockSpec((D, D), lambda i: (0, 0)),
            pl.BlockSpec((blk, 1), lambda i: (i, 0)),
            pl.BlockSpec((blk, 1), lambda i: (i, 0)),
        ],
        out_specs=pl.BlockSpec((blk, D), lambda i: (i, 0)),
        out_shape=jax.ShapeDtypeStruct((B_PAD, D), jnp.float32),
    )(x_g, W, ds0, ds1)


@functools.partial(
    pl.kernel,
    out_type=jax.ShapeDtypeStruct((NC, N_DST, D), jnp.float32),
    mesh=_mesh,
    scratch_types=[
        pltpu.VMEM((NCH, EK), jnp.int32),
        pltpu.VMEM((NCH, EK), jnp.int32),
        [pltpu.VMEM((EK, D), jnp.float32) for _ in range(6)],
        [pltpu.SemaphoreType.DMA for _ in range(6)],
        [pltpu.SemaphoreType.DMA for _ in range(6)],
        pltpu.SemaphoreType.DMA,
        pltpu.SemaphoreType.DMA,
        pltpu.VMEM_SHARED((AGR, D), jnp.float32),
    ],
)
def _aggregate(h_hbm, esrc_hbm, edst_hbm, zero_hbm, out_hbm,
               sidx_v, didx_v, rows_bufs, gsems, ssems, isem0, isem1, agg_sh):
    cid = lax.axis_index("c")
    sid = lax.axis_index("s")
    wid = _wid()
    rpt = AGR // NS
    ept = N_DST // NS

    id0 = pltpu.async_copy(esrc_hbm.at[wid], sidx_v, isem0)
    id1 = pltpu.async_copy(edst_hbm.at[wid], didx_v, isem1)
    pltpu.sync_copy(zero_hbm.at[pl.ds(sid * rpt, rpt), :],
                    agg_sh.at[pl.ds(sid * rpt, rpt), :])
    id0.wait()
    id1.wait()
    plsc.subcore_barrier()

    def fire_gather(j, b):
        return pltpu.async_copy(h_hbm.at[sidx_v.at[j]], rows_bufs[b],
                                gsems[b])

    def wait_gather(j, b):
        pltpu.make_async_copy(h_hbm.at[sidx_v.at[j]], rows_bufs[b],
                              gsems[b]).wait()

    def fire_scatter(j, b):
        return pltpu.async_copy(rows_bufs[b], agg_sh.at[didx_v.at[j]],
                                ssems[b], add=True)

    def wait_scatter(j, b):
        pltpu.make_async_copy(rows_bufs[b], agg_sh.at[didx_v.at[j]],
                              ssems[b]).wait()

    for b in range(3):
        fire_gather(b, b)

    NB = 20

    def body(g, carry):
        j0 = 6 * g
        for b in range(3):
            fire_gather(j0 + 3 + b, 3 + b)
        for b in range(3):
            wait_gather(j0 + b, b)
            fire_scatter(j0 + b, b)
        for b in range(3):
            wait_scatter(j0 + b, b)
            fire_gather(j0 + 6 + b, b)
        for b in range(3):
            wait_gather(j0 + 3 + b, 3 + b)
            fire_scatter(j0 + 3 + b, 3 + b)
        for b in range(3):
            wait_scatter(j0 + 3 + b, 3 + b)
        return carry

    lax.fori_loop(0, NB, body, None)
    j0 = 6 * NB
    for b in range(3):
        wait_gather(j0 + b, b)
        fire_scatter(j0 + b, b)
    fire_gather(j0 + 3, 3)
    fire_gather(j0 + 4, 4)
    for b in range(2):
        wait_gather(j0 + 3 + b, 3 + b)
        fire_scatter(j0 + 3 + b, 3 + b)
    for b in range(5):
        wait_scatter(j0 + b, b)

    plsc.subcore_barrier()
    pltpu.sync_copy(agg_sh.at[pl.ds(sid * ept, ept), :],
                    out_hbm.at[cid, pl.ds(sid * ept, ept), :])


def _final_body(p0_ref, p1_ref, d0_ref, d1_ref, b_ref, o_ref):
    g = lax.rsqrt(jnp.maximum(d0_ref[...] + d1_ref[...], 1.0))
    a = (p0_ref[...] + p1_ref[...]) * g + b_ref[...]
    o_ref[...] = jnp.where(a > 0, a, jnp.exp(jnp.minimum(a, 0.0)) - 1.0)


def _finalize(p0, p1, dd0, dd1, b2):
    return pl.pallas_call(
        _final_body,
        grid=(1,),
        in_specs=[
            pl.BlockSpec((N_DST, D), lambda i: (0, 0)),
            pl.BlockSpec((N_DST, D), lambda i: (0, 0)),
            pl.BlockSpec((N_DST, 1), lambda i: (0, 0)),
            pl.BlockSpec((N_DST, 1), lambda i: (0, 0)),
            pl.BlockSpec((1, D), lambda i: (0, 0)),
        ],
        out_specs=pl.BlockSpec((N_DST, D), lambda i: (0, 0)),
        out_shape=jax.ShapeDtypeStruct((N_DST, D), jnp.float32),
    )(p0, p1, dd0, dd1, b2)


def kernel(x, n_id, res_n_id, edge_src, edge_dst, W, b):
    del res_n_id
    nid_pad = jnp.concatenate(
        [n_id, jnp.zeros((B_PAD - N_SRC,), jnp.int32)])
    esrc_r = edge_src.reshape(NW, NCH, EK)
    edst_r = edge_dst.reshape(NW, NCH, EK)

    x_g, hist0, hist1 = _gather_hist(x, nid_pad, edge_src, edge_dst)

    ds0 = hist0[:B_PAD].reshape(B_PAD, 1)
    ds1 = hist1[:B_PAD].reshape(B_PAD, 1)
    h = _matmul(x_g, W, ds0, ds1)

    zeros2d = jnp.zeros((AGR, D), jnp.float32)
    parts = _aggregate(h, esrc_r, edst_r, zeros2d)

    dd0 = hist0[N_SRC:N_SRC + N_DST].reshape(N_DST, 1)
    dd1 = hist1[N_SRC:N_SRC + N_DST].reshape(N_DST, 1)
    return _finalize(parts[0], parts[1], dd0, dd1, b.reshape(1, D))


# speedup vs baseline: 1.9885x; 1.0072x over previous
import functools

import jax
import jax.numpy as jnp
from jax import lax
from jax.experimental import pallas as pl
from jax.experimental.pallas import tpu as pltpu
from jax.experimental.pallas import tpu_sc as plsc

N_SRC = 10000
N_DST = 2048
E = 320000
D = 128

NC = 2
NS = 16
NW = NC * NS

B_PAD = 10240
GPW = B_PAD // NW
GK = 80

EPW = E // NW
EK = 100
NCH = EPW // EK
AGR = N_DST

HTOT = 12288
SW = HTOT // NS

_mesh = plsc.VectorSubcoreMesh(core_axis_name="c", subcore_axis_name="s")


def _wid():
    return lax.axis_index("s") * NC + lax.axis_index("c")


@functools.partial(
    pl.kernel,
    out_type=[jax.ShapeDtypeStruct((B_PAD, D), jnp.float32),
              jax.ShapeDtypeStruct((HTOT,), jnp.float32),
              jax.ShapeDtypeStruct((HTOT,), jnp.float32)],
    mesh=_mesh,
    scratch_types=[
        pltpu.VMEM((GPW,), jnp.int32),
        [pltpu.VMEM((GK, D), jnp.float32) for _ in range(GPW // GK)],
        [pltpu.SemaphoreType.DMA for _ in range(GPW // GK)],
        pltpu.VMEM((EPW,), jnp.int32),
        pltpu.VMEM((EPW,), jnp.int32),
        pltpu.SemaphoreType.DMA,
        pltpu.SemaphoreType.DMA,
        pltpu.VMEM((HTOT,), jnp.float32),
        pltpu.VMEM((SW,), jnp.float32),
        pltpu.VMEM((NS, SW), jnp.float32),
        pltpu.VMEM_SHARED((NS, HTOT), jnp.float32),
    ],
    compiler_params=pltpu.CompilerParams(needs_layout_passes=False),
)
def _gather_hist(x_hbm, nid_hbm, esrc_hbm, edst_hbm,
                 out_hbm, out0_hbm, out1_hbm,
                 gidx_v, rows_bufs, gsems, sidx_v, didx_v, esem0, esem1,
                 hist_v, acc_v, tbuf_v, hist_sh):
    cid = lax.axis_index("c")
    sid = lax.axis_index("s")
    wid = _wid()
    zeros16 = jnp.zeros((16,), jnp.float32)
    ones16 = jnp.ones((16,), jnp.float32)
    nchunk = GPW // GK
    base = wid * GPW

    pltpu.sync_copy(nid_hbm.at[pl.ds(base, GPW)], gidx_v)
    gds = [pltpu.async_copy(x_hbm.at[gidx_v.at[pl.ds(j * GK, GK)]],
                            rows_bufs[j], gsems[j])
           for j in range(nchunk)]
    ed0 = pltpu.async_copy(esrc_hbm.at[pl.ds(wid * EPW, EPW)], sidx_v, esem0)
    ed1 = pltpu.async_copy(edst_hbm.at[pl.ds(wid * EPW, EPW)], didx_v, esem1)

    def zero_body(j, carry):
        hist_v[pl.ds(pl.multiple_of(j * 16, 16), 16)] = zeros16
        return carry

    lax.fori_loop(0, HTOT // 16, zero_body, None)

    for j in range(nchunk):
        gds[j].wait()
        pltpu.sync_copy(rows_bufs[j], out_hbm.at[pl.ds(base + j * GK, GK)])
    ed0.wait()
    ed1.wait()

    def scat_body(i, carry):
        sl = pl.ds(pl.multiple_of(i * 16, 16), 16)
        plsc.addupdate_scatter(hist_v, [sidx_v[sl]], ones16)
        plsc.addupdate_scatter(hist_v, [didx_v[sl] + N_SRC], ones16)
        return carry

    lax.fori_loop(0, EPW // 16, scat_body, None)

    pltpu.sync_copy(hist_v, hist_sh.at[sid])
    plsc.subcore_barrier()
    pltpu.sync_copy(hist_sh.at[:, pl.ds(sid * SW, SW)], tbuf_v)

    def red_body(j, carry):
        sl = pl.ds(pl.multiple_of(j * 16, 16), 16)
        a = tbuf_v[0, sl]
        for t in range(1, NS):
            a = a + tbuf_v[t, sl]
        acc_v[sl] = a
        return carry

    lax.fori_loop(0, SW // 16, red_body, None)

    @pl.when(cid == 0)
    def _():
        pltpu.sync_copy(acc_v, out0_hbm.at[pl.ds(sid * SW, SW)])

    @pl.when(cid == 1)
    def _():
        pltpu.sync_copy(acc_v, out1_hbm.at[pl.ds(sid * SW, SW)])


def _matmul_body(x_ref, w_ref, d0_ref, d1_ref, o_ref):
    deg = d0_ref[...] + d1_ref[...]
    f = lax.rsqrt(jnp.maximum(deg, 1.0))
    h = jnp.dot(x_ref[...], w_ref[...], preferred_element_type=jnp.float32,
                precision=lax.Precision.HIGHEST)
    o_ref[...] = h * f


def _matmul(x_g, W, ds0, ds1):
    blk = 512
    grid = B_PAD // blk
    return pl.pallas_call(
        _matmul_body,
        grid=(grid,),
        in_specs=[
            pl.BlockSpec((blk, D), lambda i: (i, 0)),
            pl.BlockSpec((D, D), lambda i: (0, 0)),
            pl.BlockSpec((blk, 1), lambda i: (i, 0)),
            pl.BlockSpec((blk, 1), lambda i: (i, 0)),
        ],
        out_specs=pl.BlockSpec((blk, D), lambda i: (i, 0)),
        out_shape=jax.ShapeDtypeStruct((B_PAD, D), jnp.float32),
    )(x_g, W, ds0, ds1)


@functools.partial(
    pl.kernel,
    out_type=jax.ShapeDtypeStruct((NC, N_DST, D), jnp.float32),
    mesh=_mesh,
    scratch_types=[
        pltpu.VMEM((NCH, EK), jnp.int32),
        pltpu.VMEM((NCH, EK), jnp.int32),
        [pltpu.VMEM((EK, D), jnp.float32) for _ in range(6)],
        [pltpu.SemaphoreType.DMA for _ in range(6)],
        [pltpu.SemaphoreType.DMA for _ in range(6)],
        pltpu.SemaphoreType.DMA,
        pltpu.SemaphoreType.DMA,
        pltpu.VMEM_SHARED((AGR, D), jnp.float32),
    ],
)
def _aggregate(h_hbm, esrc_hbm, edst_hbm, zero_hbm, out_hbm,
               sidx_v, didx_v, rows_bufs, gsems, ssems, isem0, isem1, agg_sh):
    cid = lax.axis_index("c")
    sid = lax.axis_index("s")
    wid = _wid()
    rpt = AGR // NS
    ept = N_DST // NS

    id0 = pltpu.async_copy(esrc_hbm.at[wid], sidx_v, isem0)
    id1 = pltpu.async_copy(edst_hbm.at[wid], didx_v, isem1)
    pltpu.sync_copy(zero_hbm.at[pl.ds(sid * rpt, rpt), :],
                    agg_sh.at[pl.ds(sid * rpt, rpt), :])
    id0.wait()
    id1.wait()
    plsc.subcore_barrier()

    def fire_gather(j, b):
        return pltpu.async_copy(h_hbm.at[sidx_v.at[j]], rows_bufs[b],
                                gsems[b])

    def wait_gather(j, b):
        pltpu.make_async_copy(h_hbm.at[sidx_v.at[j]], rows_bufs[b],
                              gsems[b]).wait()

    def fire_scatter(j, b):
        return pltpu.async_copy(rows_bufs[b], agg_sh.at[didx_v.at[j]],
                                ssems[b], add=True)

    def wait_scatter(j, b):
        pltpu.make_async_copy(rows_bufs[b], agg_sh.at[didx_v.at[j]],
                              ssems[b]).wait()

    for b in range(3):
        fire_gather(b, b)

    NB = 16

    def body(g, carry):
        j0 = 6 * g
        for b in range(3):
            fire_gather(j0 + 3 + b, 3 + b)
        for b in range(3):
            wait_gather(j0 + b, b)
            fire_scatter(j0 + b, b)
        for b in range(3):
            wait_scatter(j0 + b, b)
            fire_gather(j0 + 6 + b, b)
        for b in range(3):
            wait_gather(j0 + 3 + b, 3 + b)
            fire_scatter(j0 + 3 + b, 3 + b)
        for b in range(3):
            wait_scatter(j0 + 3 + b, 3 + b)
        return carry

    lax.fori_loop(0, NB, body, None)
    j0 = 6 * NB
    fire_gather(j0 + 3, 3)
    for b in range(3):
        wait_gather(j0 + b, b)
        fire_scatter(j0 + b, b)
    wait_gather(j0 + 3, 3)
    fire_scatter(j0 + 3, 3)
    for b in range(4):
        wait_scatter(j0 + b, b)

    plsc.subcore_barrier()
    pltpu.sync_copy(agg_sh.at[pl.ds(sid * ept, ept), :],
                    out_hbm.at[cid, pl.ds(sid * ept, ept), :])


def _final_body(p0_ref, p1_ref, d0_ref, d1_ref, b_ref, o_ref):
    g = lax.rsqrt(jnp.maximum(d0_ref[...] + d1_ref[...], 1.0))
    a = (p0_ref[...] + p1_ref[...]) * g + b_ref[...]
    o_ref[...] = jnp.where(a > 0, a, jnp.exp(jnp.minimum(a, 0.0)) - 1.0)


def _finalize(p0, p1, dd0, dd1, b2):
    return pl.pallas_call(
        _final_body,
        grid=(1,),
        in_specs=[
            pl.BlockSpec((N_DST, D), lambda i: (0, 0)),
            pl.BlockSpec((N_DST, D), lambda i: (0, 0)),
            pl.BlockSpec((N_DST, 1), lambda i: (0, 0)),
            pl.BlockSpec((N_DST, 1), lambda i: (0, 0)),
            pl.BlockSpec((1, D), lambda i: (0, 0)),
        ],
        out_specs=pl.BlockSpec((N_DST, D), lambda i: (0, 0)),
        out_shape=jax.ShapeDtypeStruct((N_DST, D), jnp.float32),
    )(p0, p1, dd0, dd1, b2)


def kernel(x, n_id, res_n_id, edge_src, edge_dst, W, b):
    del res_n_id
    nid_pad = jnp.concatenate(
        [n_id, jnp.zeros((B_PAD - N_SRC,), jnp.int32)])
    esrc_r = edge_src.reshape(NW, NCH, EK)
    edst_r = edge_dst.reshape(NW, NCH, EK)

    x_g, hist0, hist1 = _gather_hist(x, nid_pad, edge_src, edge_dst)

    ds0 = hist0[:B_PAD].reshape(B_PAD, 1)
    ds1 = hist1[:B_PAD].reshape(B_PAD, 1)
    h = _matmul(x_g, W, ds0, ds1)

    zeros2d = jnp.zeros((AGR, D), jnp.float32)
    parts = _aggregate(h, esrc_r, edst_r, zeros2d)

    dd0 = hist0[N_SRC:N_SRC + N_DST].reshape(N_DST, 1)
    dd1 = hist1[N_SRC:N_SRC + N_DST].reshape(N_DST, 1)
    return _finalize(parts[0], parts[1], dd0, dd1, b.reshape(1, D))


# speedup vs baseline: 2.1616x; 1.0870x over previous
import functools

import jax
import jax.numpy as jnp
from jax import lax
from jax.experimental import pallas as pl
from jax.experimental.pallas import tpu as pltpu
from jax.experimental.pallas import tpu_sc as plsc

N_SRC = 10000
N_DST = 2048
E = 320000
D = 128

NC = 2
NS = 16
NW = NC * NS

B_PAD = 10240
GPW = B_PAD // NW
GK = 80

EPW = E // NW
EK = 100
NCH = EPW // EK
AGR = N_DST

HTOT = 12288
DOFF = B_PAD
SW = HTOT // NS

_mesh = plsc.VectorSubcoreMesh(core_axis_name="c", subcore_axis_name="s")


def _wid():
    return lax.axis_index("s") * NC + lax.axis_index("c")


@functools.partial(
    pl.kernel,
    out_type=[jax.ShapeDtypeStruct((B_PAD, D), jnp.float32),
              jax.ShapeDtypeStruct((HTOT,), jnp.float32),
              jax.ShapeDtypeStruct((HTOT,), jnp.float32)],
    mesh=_mesh,
    scratch_types=[
        pltpu.VMEM((GPW,), jnp.int32),
        [pltpu.VMEM((GK, D), jnp.float32) for _ in range(GPW // GK)],
        [pltpu.SemaphoreType.DMA for _ in range(GPW // GK)],
        pltpu.VMEM((EPW,), jnp.int32),
        pltpu.VMEM((EPW,), jnp.int32),
        pltpu.SemaphoreType.DMA,
        pltpu.SemaphoreType.DMA,
        pltpu.VMEM((HTOT,), jnp.float32),
        pltpu.VMEM((SW,), jnp.float32),
        pltpu.VMEM((NS, SW), jnp.float32),
        pltpu.VMEM_SHARED((NS, HTOT), jnp.float32),
    ],
    compiler_params=pltpu.CompilerParams(needs_layout_passes=False),
)
def _gather_hist(x_hbm, nid_hbm, esrc_hbm, edst_hbm,
                 out_hbm, out0_hbm, out1_hbm,
                 gidx_v, rows_bufs, gsems, sidx_v, didx_v, esem0, esem1,
                 hist_v, acc_v, tbuf_v, hist_sh):
    cid = lax.axis_index("c")
    sid = lax.axis_index("s")
    wid = _wid()
    zeros16 = jnp.zeros((16,), jnp.float32)
    ones16 = jnp.ones((16,), jnp.float32)
    nchunk = GPW // GK
    base = wid * GPW

    for j in range(nchunk):

        @pl.when(base + (j + 1) * GK <= N_SRC)
        def _():
            pltpu.sync_copy(nid_hbm.at[pl.ds(base + j * GK, GK)],
                            gidx_v.at[pl.ds(j * GK, GK)])

    for j in range(nchunk):

        @pl.when(base + (j + 1) * GK <= N_SRC)
        def _():
            pltpu.async_copy(x_hbm.at[gidx_v.at[pl.ds(j * GK, GK)]],
                             rows_bufs[j], gsems[j])
    ed0 = pltpu.async_copy(esrc_hbm.at[pl.ds(wid * EPW, EPW)], sidx_v, esem0)
    ed1 = pltpu.async_copy(edst_hbm.at[pl.ds(wid * EPW, EPW)], didx_v, esem1)

    def zero_body(j, carry):
        hist_v[pl.ds(pl.multiple_of(j * 16, 16), 16)] = zeros16
        return carry

    lax.fori_loop(0, HTOT // 16, zero_body, None)

    for j in range(nchunk):

        @pl.when(base + (j + 1) * GK <= N_SRC)
        def _():
            pltpu.make_async_copy(x_hbm.at[gidx_v.at[pl.ds(j * GK, GK)]],
                                  rows_bufs[j], gsems[j]).wait()
            pltpu.sync_copy(rows_bufs[j],
                            out_hbm.at[pl.ds(base + j * GK, GK)])
    ed0.wait()
    ed1.wait()

    def scat_body(i, carry):
        sl = pl.ds(pl.multiple_of(i * 16, 16), 16)
        plsc.addupdate_scatter(hist_v, [sidx_v[sl]], ones16)
        plsc.addupdate_scatter(hist_v, [didx_v[sl] + DOFF], ones16)
        return carry

    lax.fori_loop(0, EPW // 16, scat_body, None)

    pltpu.sync_copy(hist_v, hist_sh.at[sid])
    plsc.subcore_barrier()
    pltpu.sync_copy(hist_sh.at[:, pl.ds(sid * SW, SW)], tbuf_v)

    def red_body(j, carry):
        sl = pl.ds(pl.multiple_of(j * 16, 16), 16)
        a = tbuf_v[0, sl]
        for t in range(1, NS):
            a = a + tbuf_v[t, sl]
        acc_v[sl] = a
        return carry

    lax.fori_loop(0, SW // 16, red_body, None)

    @pl.when(cid == 0)
    def _():
        pltpu.sync_copy(acc_v, out0_hbm.at[pl.ds(sid * SW, SW)])

    @pl.when(cid == 1)
    def _():
        pltpu.sync_copy(acc_v, out1_hbm.at[pl.ds(sid * SW, SW)])


def _matmul_body(x_ref, w_ref, d0_ref, d1_ref, o_ref):
    deg = d0_ref[...] + d1_ref[...]
    f = lax.rsqrt(jnp.maximum(deg, 1.0))
    h = jnp.dot(x_ref[...], w_ref[...], preferred_element_type=jnp.float32)
    o_ref[...] = h * f


def _matmul(x_g, W, ds0, ds1):
    blk = 512
    grid = B_PAD // blk
    return pl.pallas_call(
        _matmul_body,
        grid=(grid,),
        in_specs=[
            pl.BlockSpec((blk, D), lambda i: (i, 0)),
            pl.BlockSpec((D, D), lambda i: (0, 0)),
            pl.BlockSpec((blk, 1), lambda i: (i, 0)),
            pl.BlockSpec((blk, 1), lambda i: (i, 0)),
        ],
        out_specs=pl.BlockSpec((blk, D), lambda i: (i, 0)),
        out_shape=jax.ShapeDtypeStruct((B_PAD, D), jnp.float32),
    )(x_g, W, ds0, ds1)


@functools.partial(
    pl.kernel,
    out_type=jax.ShapeDtypeStruct((NC, N_DST, D), jnp.float32),
    mesh=_mesh,
    scratch_types=[
        pltpu.VMEM((NCH, EK), jnp.int32),
        pltpu.VMEM((NCH, EK), jnp.int32),
        [pltpu.VMEM((EK, D), jnp.float32) for _ in range(6)],
        [pltpu.SemaphoreType.DMA for _ in range(6)],
        [pltpu.SemaphoreType.DMA for _ in range(6)],
        pltpu.SemaphoreType.DMA,
        pltpu.SemaphoreType.DMA,
        pltpu.VMEM_SHARED((AGR, D), jnp.float32),
    ],
)
def _aggregate(h_hbm, esrc_hbm, edst_hbm, zero_hbm, out_hbm,
               sidx_v, didx_v, rows_bufs, gsems, ssems, isem0, isem1, agg_sh):
    cid = lax.axis_index("c")
    sid = lax.axis_index("s")
    wid = _wid()
    rpt = AGR // NS
    ept = N_DST // NS

    id0 = pltpu.async_copy(esrc_hbm.at[wid], sidx_v, isem0)
    id1 = pltpu.async_copy(edst_hbm.at[wid], didx_v, isem1)
    pltpu.sync_copy(zero_hbm.at[pl.ds(sid * rpt, rpt), :],
                    agg_sh.at[pl.ds(sid * rpt, rpt), :])
    id0.wait()
    id1.wait()
    plsc.subcore_barrier()

    def fire_gather(j, b):
        return pltpu.async_copy(h_hbm.at[sidx_v.at[j]], rows_bufs[b],
                                gsems[b])

    def wait_gather(j, b):
        pltpu.make_async_copy(h_hbm.at[sidx_v.at[j]], rows_bufs[b],
                              gsems[b]).wait()

    def fire_scatter(j, b):
        return pltpu.async_copy(rows_bufs[b], agg_sh.at[didx_v.at[j]],
                                ssems[b], add=True)

    def wait_scatter(j, b):
        pltpu.make_async_copy(rows_bufs[b], agg_sh.at[didx_v.at[j]],
                              ssems[b]).wait()

    for b in range(3):
        fire_gather(b, b)

    NB = 16

    def body(g, carry):
        j0 = 6 * g
        for b in range(3):
            fire_gather(j0 + 3 + b, 3 + b)
        for b in range(3):
            wait_gather(j0 + b, b)
            fire_scatter(j0 + b, b)
        for b in range(3):
            wait_scatter(j0 + b, b)
            fire_gather(j0 + 6 + b, b)
        for b in range(3):
            wait_gather(j0 + 3 + b, 3 + b)
            fire_scatter(j0 + 3 + b, 3 + b)
        for b in range(3):
            wait_scatter(j0 + 3 + b, 3 + b)
        return carry

    lax.fori_loop(0, NB, body, None)
    j0 = 6 * NB
    fire_gather(j0 + 3, 3)
    for b in range(3):
        wait_gather(j0 + b, b)
        fire_scatter(j0 + b, b)
    wait_gather(j0 + 3, 3)
    fire_scatter(j0 + 3, 3)
    for b in range(4):
        wait_scatter(j0 + b, b)

    plsc.subcore_barrier()
    pltpu.sync_copy(agg_sh.at[pl.ds(sid * ept, ept), :],
                    out_hbm.at[cid, pl.ds(sid * ept, ept), :])


def _final_body(p0_ref, p1_ref, d0_ref, d1_ref, b_ref, o_ref):
    g = lax.rsqrt(jnp.maximum(d0_ref[...] + d1_ref[...], 1.0))
    a = (p0_ref[0] + p1_ref[0]) * g + b_ref[...]
    o_ref[...] = jnp.where(a > 0, a, jnp.exp(jnp.minimum(a, 0.0)) - 1.0)


def _finalize(parts, h0v, h1v, b2):
    return pl.pallas_call(
        _final_body,
        grid=(1,),
        in_specs=[
            pl.BlockSpec((1, N_DST, D), lambda i: (0, 0, 0)),
            pl.BlockSpec((1, N_DST, D), lambda i: (1, 0, 0)),
            pl.BlockSpec((N_DST, 1), lambda i: (DOFF // N_DST, 0)),
            pl.BlockSpec((N_DST, 1), lambda i: (DOFF // N_DST, 0)),
            pl.BlockSpec((1, D), lambda i: (0, 0)),
        ],
        out_specs=pl.BlockSpec((N_DST, D), lambda i: (0, 0)),
        out_shape=jax.ShapeDtypeStruct((N_DST, D), jnp.float32),
    )(parts, parts, h0v, h1v, b2)


def kernel(x, n_id, res_n_id, edge_src, edge_dst, W, b):
    del res_n_id
    esrc_r = edge_src.reshape(NW, NCH, EK)
    edst_r = edge_dst.reshape(NW, NCH, EK)

    x_g, hist0, hist1 = _gather_hist(x, n_id, edge_src, edge_dst)

    h0v = hist0.reshape(HTOT, 1)
    h1v = hist1.reshape(HTOT, 1)
    h = _matmul(x_g, W, h0v, h1v)

    zeros2d = jnp.zeros((AGR, D), jnp.float32)
    parts = _aggregate(h, esrc_r, edst_r, zeros2d)

    return _finalize(parts, h0v, h1v, b.reshape(1, D))


# speedup vs baseline: 2.2377x; 1.0352x over previous
import functools

import jax
import jax.numpy as jnp
from jax import lax
from jax.experimental import pallas as pl
from jax.experimental.pallas import tpu as pltpu
from jax.experimental.pallas import tpu_sc as plsc

N_SRC = 10000
N_DST = 2048
E = 320000
D = 128

NC = 2
NS = 16
NW = NC * NS

B_PAD = 10240
GPW = B_PAD // NW
GK = 80

EPW = E // NW
EK = 80
NCH = EPW // EK
AGR = N_DST

HTOT = 12288
DOFF = B_PAD
SW = HTOT // NS

_mesh = plsc.VectorSubcoreMesh(core_axis_name="c", subcore_axis_name="s")


def _wid():
    return lax.axis_index("s") * NC + lax.axis_index("c")


@functools.partial(
    pl.kernel,
    out_type=[jax.ShapeDtypeStruct((B_PAD, D), jnp.float32),
              jax.ShapeDtypeStruct((HTOT,), jnp.float32),
              jax.ShapeDtypeStruct((HTOT,), jnp.float32)],
    mesh=_mesh,
    scratch_types=[
        pltpu.VMEM((GPW,), jnp.int32),
        [pltpu.VMEM((GK, D), jnp.float32) for _ in range(GPW // GK)],
        [pltpu.SemaphoreType.DMA for _ in range(GPW // GK)],
        pltpu.VMEM((EPW,), jnp.int32),
        pltpu.VMEM((EPW,), jnp.int32),
        pltpu.SemaphoreType.DMA,
        pltpu.SemaphoreType.DMA,
        pltpu.VMEM((HTOT,), jnp.float32),
        pltpu.VMEM((SW,), jnp.float32),
        pltpu.VMEM((NS, SW), jnp.float32),
        pltpu.VMEM_SHARED((NS, HTOT), jnp.float32),
    ],
    compiler_params=pltpu.CompilerParams(needs_layout_passes=False),
)
def _gather_hist(x_hbm, nid_hbm, esrc_hbm, edst_hbm,
                 out_hbm, out0_hbm, out1_hbm,
                 gidx_v, rows_bufs, gsems, sidx_v, didx_v, esem0, esem1,
                 hist_v, acc_v, tbuf_v, hist_sh):
    cid = lax.axis_index("c")
    sid = lax.axis_index("s")
    wid = _wid()
    zeros16 = jnp.zeros((16,), jnp.float32)
    ones16 = jnp.ones((16,), jnp.float32)
    nchunk = GPW // GK
    base = wid * GPW

    for j in range(nchunk):

        @pl.when(base + (j + 1) * GK <= N_SRC)
        def _():
            pltpu.sync_copy(nid_hbm.at[pl.ds(base + j * GK, GK)],
                            gidx_v.at[pl.ds(j * GK, GK)])

    for j in range(nchunk):

        @pl.when(base + (j + 1) * GK <= N_SRC)
        def _():
            pltpu.async_copy(x_hbm.at[gidx_v.at[pl.ds(j * GK, GK)]],
                             rows_bufs[j], gsems[j])
    ed0 = pltpu.async_copy(esrc_hbm.at[pl.ds(wid * EPW, EPW)], sidx_v, esem0)
    ed1 = pltpu.async_copy(edst_hbm.at[pl.ds(wid * EPW, EPW)], didx_v, esem1)

    def zero_body(j, carry):
        hist_v[pl.ds(pl.multiple_of(j * 16, 16), 16)] = zeros16
        return carry

    lax.fori_loop(0, HTOT // 16, zero_body, None)

    for j in range(nchunk):

        @pl.when(base + (j + 1) * GK <= N_SRC)
        def _():
            pltpu.make_async_copy(x_hbm.at[gidx_v.at[pl.ds(j * GK, GK)]],
                                  rows_bufs[j], gsems[j]).wait()
            pltpu.sync_copy(rows_bufs[j],
                            out_hbm.at[pl.ds(base + j * GK, GK)])
    ed0.wait()
    ed1.wait()

    def scat_body(i, carry):
        sl = pl.ds(pl.multiple_of(i * 16, 16), 16)
        plsc.addupdate_scatter(hist_v, [sidx_v[sl]], ones16)
        plsc.addupdate_scatter(hist_v, [didx_v[sl] + DOFF], ones16)
        return carry

    lax.fori_loop(0, EPW // 16, scat_body, None)

    pltpu.sync_copy(hist_v, hist_sh.at[sid])
    plsc.subcore_barrier()
    pltpu.sync_copy(hist_sh.at[:, pl.ds(sid * SW, SW)], tbuf_v)

    def red_body(j, carry):
        sl = pl.ds(pl.multiple_of(j * 16, 16), 16)
        a = tbuf_v[0, sl]
        for t in range(1, NS):
            a = a + tbuf_v[t, sl]
        acc_v[sl] = a
        return carry

    lax.fori_loop(0, SW // 16, red_body, None)

    @pl.when(cid == 0)
    def _():
        pltpu.sync_copy(acc_v, out0_hbm.at[pl.ds(sid * SW, SW)])

    @pl.when(cid == 1)
    def _():
        pltpu.sync_copy(acc_v, out1_hbm.at[pl.ds(sid * SW, SW)])


def _matmul_body(x_ref, w_ref, d0_ref, d1_ref, o_ref):
    deg = d0_ref[...] + d1_ref[...]
    f = lax.rsqrt(jnp.maximum(deg, 1.0))
    h = jnp.dot(x_ref[...], w_ref[...], preferred_element_type=jnp.float32)
    o_ref[...] = h * f


def _matmul(x_g, W, ds0, ds1):
    blk = 1024
    grid = B_PAD // blk
    return pl.pallas_call(
        _matmul_body,
        grid=(grid,),
        in_specs=[
            pl.BlockSpec((blk, D), lambda i: (i, 0)),
            pl.BlockSpec((D, D), lambda i: (0, 0)),
            pl.BlockSpec((blk, 1), lambda i: (i, 0)),
            pl.BlockSpec((blk, 1), lambda i: (i, 0)),
        ],
        out_specs=pl.BlockSpec((blk, D), lambda i: (i, 0)),
        out_shape=jax.ShapeDtypeStruct((B_PAD, D), jnp.float32),
    )(x_g, W, ds0, ds1)


@functools.partial(
    pl.kernel,
    out_type=jax.ShapeDtypeStruct((NC, N_DST, D), jnp.float32),
    mesh=_mesh,
    scratch_types=[
        pltpu.VMEM((EPW,), jnp.int32),
        pltpu.VMEM((EPW,), jnp.int32),
        [pltpu.VMEM((EK, D), jnp.float32) for _ in range(6)],
        [pltpu.VMEM((EK,), jnp.int32) for _ in range(6)],
        [pltpu.SemaphoreType.DMA for _ in range(6)],
        [pltpu.SemaphoreType.DMA for _ in range(6)],
        pltpu.SemaphoreType.DMA,
        pltpu.SemaphoreType.DMA,
        pltpu.VMEM((AGR // NS, D), jnp.float32),
        pltpu.VMEM_SHARED((AGR, D), jnp.float32),
    ],
    compiler_params=pltpu.CompilerParams(needs_layout_passes=False),
)
def _aggregate(h_hbm, esrc_hbm, edst_hbm, out_hbm,
               sidx_v, didx_v, rows_bufs, didx_bufs, gsems, ssems,
               isem0, isem1, zrow_v, agg_sh):
    cid = lax.axis_index("c")
    sid = lax.axis_index("s")
    wid = _wid()
    rpt = AGR // NS
    ept = N_DST // NS

    id0 = pltpu.async_copy(esrc_hbm.at[pl.ds(wid * EPW, EPW)], sidx_v, isem0)
    id1 = pltpu.async_copy(edst_hbm.at[pl.ds(wid * EPW, EPW)], didx_v, isem1)

    zeros16 = jnp.zeros((16,), jnp.float32)

    def zrow_body2(i, carry):
        for k in range(D // 16):
            zrow_v[i, pl.ds(k * 16, 16)] = zeros16
        return carry

    lax.fori_loop(0, rpt, zrow_body2, None)
    pltpu.sync_copy(zrow_v, agg_sh.at[pl.ds(sid * rpt, rpt), :])
    id0.wait()
    id1.wait()
    plsc.subcore_barrier()

    def fire_gather(j, b):
        sl = pl.ds(pl.multiple_of(j * EK, 16), EK)
        return pltpu.async_copy(h_hbm.at[sidx_v.at[sl]], rows_bufs[b],
                                gsems[b])

    def wait_gather(j, b):
        sl = pl.ds(pl.multiple_of(j * EK, 16), EK)
        pltpu.make_async_copy(h_hbm.at[sidx_v.at[sl]], rows_bufs[b],
                              gsems[b]).wait()

    def load_didx(j, b):
        for k in range(EK // 16):
            off = pl.multiple_of(j * EK + k * 16, 16)
            didx_bufs[b][pl.ds(k * 16, 16)] = didx_v[pl.ds(off, 16)]

    def fire_scatter(j, b):
        return pltpu.async_copy(rows_bufs[b], agg_sh.at[didx_bufs[b]],
                                ssems[b], add=True)

    def wait_scatter(j, b):
        pltpu.make_async_copy(rows_bufs[b], agg_sh.at[didx_bufs[b]],
                              ssems[b]).wait()

    for b in range(3):
        fire_gather(b, b)

    NB = 20

    def body(g, carry):
        j0 = 6 * g
        for b in range(3):
            fire_gather(j0 + 3 + b, 3 + b)
        for b in range(3):
            load_didx(j0 + b, b)
            wait_gather(j0 + b, b)
            fire_scatter(j0 + b, b)
        for b in range(3):
            load_didx(j0 + 3 + b, 3 + b)
            wait_scatter(j0 + b, b)
            fire_gather(j0 + 6 + b, b)
        for b in range(3):
            wait_gather(j0 + 3 + b, 3 + b)
            fire_scatter(j0 + 3 + b, 3 + b)
        for b in range(3):
            wait_scatter(j0 + 3 + b, 3 + b)
        return carry

    lax.fori_loop(0, NB, body, None)
    j0 = 6 * NB
    fire_gather(j0 + 3, 3)
    fire_gather(j0 + 4, 4)
    for b in range(3):
        load_didx(j0 + b, b)
        wait_gather(j0 + b, b)
        fire_scatter(j0 + b, b)
    for b in range(2):
        load_didx(j0 + 3 + b, 3 + b)
        wait_gather(j0 + 3 + b, 3 + b)
        fire_scatter(j0 + 3 + b, 3 + b)
    for b in range(5):
        wait_scatter(j0 + b, b)

    plsc.subcore_barrier()
    pltpu.sync_copy(agg_sh.at[pl.ds(sid * ept, ept), :],
                    out_hbm.at[cid, pl.ds(sid * ept, ept), :])


def _final_body(p0_ref, p1_ref, d0_ref, d1_ref, b_ref, o_ref):
    g = lax.rsqrt(jnp.maximum(d0_ref[...] + d1_ref[...], 1.0))
    a = (p0_ref[0] + p1_ref[0]) * g + b_ref[...]
    o_ref[...] = jnp.where(a > 0, a, jnp.exp(jnp.minimum(a, 0.0)) - 1.0)


def _finalize(parts, h0v, h1v, b2):
    return pl.pallas_call(
        _final_body,
        grid=(1,),
        in_specs=[
            pl.BlockSpec((1, N_DST, D), lambda i: (0, 0, 0)),
            pl.BlockSpec((1, N_DST, D), lambda i: (1, 0, 0)),
            pl.BlockSpec((N_DST, 1), lambda i: (DOFF // N_DST, 0)),
            pl.BlockSpec((N_DST, 1), lambda i: (DOFF // N_DST, 0)),
            pl.BlockSpec((1, D), lambda i: (0, 0)),
        ],
        out_specs=pl.BlockSpec((N_DST, D), lambda i: (0, 0)),
        out_shape=jax.ShapeDtypeStruct((N_DST, D), jnp.float32),
    )(parts, parts, h0v, h1v, b2)


def kernel(x, n_id, res_n_id, edge_src, edge_dst, W, b):
    del res_n_id
    x_g, hist0, hist1 = _gather_hist(x, n_id, edge_src, edge_dst)

    h0v = hist0.reshape(HTOT, 1)
    h1v = hist1.reshape(HTOT, 1)
    h = _matmul(x_g, W, h0v, h1v)

    parts = _aggregate(h, edge_src, edge_dst)

    return _finalize(parts, h0v, h1v, b.reshape(1, D))
